# Initial kernel scaffold; baseline (speedup 1.0000x reference)
import jax
import jax.numpy as jnp
from jax.experimental import pallas as pl


def kernel(x_nchw, w1m, g1, b1, w2m, g2, b2):
    raise NotImplementedError("write your pallas kernel here")


# speedup vs baseline: 1.0814x; 1.0814x over previous
import jax
import jax.numpy as jnp
from jax.experimental import pallas as pl
from jax.experimental.pallas import tpu as pltpu

_BN_EPS = 1e-5
_H = 28
_W = 28
_WP = 32
_ROWS = 31 * _WP
_OUT_ROWS = _H * _WP


def _cparams():
    return pltpu.CompilerParams(
        dimension_semantics=("parallel",),
        vmem_limit_bytes=64 * 1024 * 1024,
    )


def _row_mask(shape):
    col = jax.lax.broadcasted_iota(jnp.int32, shape, 0) % _WP
    return col < _W


def _shifted_copies(xb):
    z1 = jnp.zeros((1, 128), dtype=xb.dtype)
    z2 = jnp.zeros((2, 128), dtype=xb.dtype)
    c1 = jnp.concatenate([xb[1:], z1], axis=0)
    c2 = jnp.concatenate([xb[2:], z2], axis=0)
    return [xb, c1, c2]


def _conv9(cs, w_ref):
    acc = jnp.zeros((_OUT_ROWS, 128), dtype=jnp.float32)
    for kx in range(3):
        ck = cs[kx]
        for ky in range(3):
            patch = ck[ky * _WP:ky * _WP + _OUT_ROWS]
            acc = acc + jnp.dot(patch, w_ref[kx, ky],
                                preferred_element_type=jnp.float32)
    return acc


def _conv1_kernel(xp_ref, w_ref, y_ref, ssum_ref, ssq_ref):
    cs = _shifted_copies(xp_ref[0])
    acc = _conv9(cs, w_ref)
    acc = jnp.where(_row_mask(acc.shape), acc, 0.0)
    y_ref[0] = acc.astype(jnp.bfloat16)
    ssum_ref[0] = jnp.sum(acc, axis=0, keepdims=True)
    ssq_ref[0] = jnp.sum(acc * acc, axis=0, keepdims=True)


def _conv2_kernel(y1_ref, sc_ref, sh_ref, w_ref, y2_ref, ssum_ref, ssq_ref):
    a = y1_ref[0].astype(jnp.float32) * sc_ref[...] + sh_ref[...]
    a = jnp.maximum(a, 0.0)
    a = jnp.where(_row_mask(a.shape), a, 0.0)
    ab = a.astype(jnp.bfloat16)

    def _placed(base):
        zt = jnp.zeros((base, 128), dtype=jnp.bfloat16)
        zb = jnp.zeros((_ROWS - _OUT_ROWS - base, 128), dtype=jnp.bfloat16)
        return jnp.concatenate([zt, ab, zb], axis=0)

    cs = [_placed(33), _placed(32), _placed(31)]
    acc = _conv9(cs, w_ref)
    acc = jnp.where(_row_mask(acc.shape), acc, 0.0)
    y2_ref[0] = acc.astype(jnp.bfloat16)
    ssum_ref[0] = jnp.sum(acc, axis=0, keepdims=True)
    ssq_ref[0] = jnp.sum(acc * acc, axis=0, keepdims=True)


def _out_kernel(y2_ref, sc_ref, sh_ref, xp_ref, o_ref):
    res = xp_ref[0][33:33 + _OUT_ROWS].astype(jnp.float32)
    o = y2_ref[0].astype(jnp.float32) * sc_ref[...] + sh_ref[...] + res
    o_ref[0] = jnp.maximum(o, 0.0)


def _conv1_call(xp, wc):
    N = xp.shape[0]
    flops = 2 * N * _H * _W * 9 * 128 * 128
    return pl.pallas_call(
        _conv1_kernel,
        out_shape=(jax.ShapeDtypeStruct((N, _OUT_ROWS, 128), jnp.bfloat16),
                   jax.ShapeDtypeStruct((N, 1, 128), jnp.float32),
                   jax.ShapeDtypeStruct((N, 1, 128), jnp.float32)),
        grid=(N,),
        in_specs=[pl.BlockSpec((1, _ROWS, 128), lambda n: (n, 0, 0)),
                  pl.BlockSpec((3, 3, 128, 128), lambda n: (0, 0, 0, 0))],
        out_specs=(pl.BlockSpec((1, _OUT_ROWS, 128), lambda n: (n, 0, 0)),
                   pl.BlockSpec((1, 1, 128), lambda n: (n, 0, 0)),
                   pl.BlockSpec((1, 1, 128), lambda n: (n, 0, 0))),
        compiler_params=_cparams(),
        cost_estimate=pl.CostEstimate(
            flops=flops, transcendentals=0,
            bytes_accessed=2 * (xp.size + wc.size) + 2 * N * _OUT_ROWS * 128),
    )(xp, wc)


def _conv2_call(y1, sc, sh, wc):
    N = y1.shape[0]
    flops = 2 * N * _H * _W * 9 * 128 * 128
    return pl.pallas_call(
        _conv2_kernel,
        out_shape=(jax.ShapeDtypeStruct((N, _OUT_ROWS, 128), jnp.bfloat16),
                   jax.ShapeDtypeStruct((N, 1, 128), jnp.float32),
                   jax.ShapeDtypeStruct((N, 1, 128), jnp.float32)),
        grid=(N,),
        in_specs=[pl.BlockSpec((1, _OUT_ROWS, 128), lambda n: (n, 0, 0)),
                  pl.BlockSpec((1, 128), lambda n: (0, 0)),
                  pl.BlockSpec((1, 128), lambda n: (0, 0)),
                  pl.BlockSpec((3, 3, 128, 128), lambda n: (0, 0, 0, 0))],
        out_specs=(pl.BlockSpec((1, _OUT_ROWS, 128), lambda n: (n, 0, 0)),
                   pl.BlockSpec((1, 1, 128), lambda n: (n, 0, 0)),
                   pl.BlockSpec((1, 1, 128), lambda n: (n, 0, 0))),
        compiler_params=_cparams(),
        cost_estimate=pl.CostEstimate(
            flops=flops, transcendentals=0,
            bytes_accessed=4 * N * _OUT_ROWS * 128),
    )(y1, sc, sh, wc)


def _out_call(y2, sc, sh, xp):
    N = y2.shape[0]
    return pl.pallas_call(
        _out_kernel,
        out_shape=jax.ShapeDtypeStruct((N, _OUT_ROWS, 128), jnp.float32),
        grid=(N,),
        in_specs=[pl.BlockSpec((1, _OUT_ROWS, 128), lambda n: (n, 0, 0)),
                  pl.BlockSpec((1, 128), lambda n: (0, 0)),
                  pl.BlockSpec((1, 128), lambda n: (0, 0)),
                  pl.BlockSpec((1, _ROWS, 128), lambda n: (n, 0, 0))],
        out_specs=pl.BlockSpec((1, _OUT_ROWS, 128), lambda n: (n, 0, 0)),
        compiler_params=_cparams(),
        cost_estimate=pl.CostEstimate(
            flops=4 * N * _OUT_ROWS * 128, transcendentals=0,
            bytes_accessed=10 * N * _OUT_ROWS * 128),
    )(y2, sc, sh, xp)


def _stats_to_affine(ssum, ssq, g, b, m):
    total = jnp.sum(ssum[:, 0, :], axis=0)
    totsq = jnp.sum(ssq[:, 0, :], axis=0)
    mean = total / m
    var = totsq / m - mean * mean
    scale = g * jax.lax.rsqrt(var + _BN_EPS)
    shift = b - mean * scale
    return scale.reshape(1, 128), shift.reshape(1, 128)


@jax.jit
def _forward(x_nchw, w1m, g1, b1, w2m, g2, b2):
    N, C, H, W = x_nchw.shape
    M = N * H * W

    w1c = jnp.transpose(w1m.reshape(3, 3, C, C), (1, 0, 2, 3)).astype(jnp.bfloat16)
    w2c = jnp.transpose(w2m.reshape(3, 3, C, C), (1, 0, 2, 3)).astype(jnp.bfloat16)

    x_nhwc = jnp.transpose(x_nchw, (0, 2, 3, 1))
    xp = jnp.pad(x_nhwc, ((0, 0), (1, 2), (1, 3), (0, 0)))
    xp = xp.astype(jnp.bfloat16).reshape(N, _ROWS, C)

    y1, s1, q1 = _conv1_call(xp, w1c)
    sc1, sh1 = _stats_to_affine(s1, q1, g1, b1, M)

    y2, s2, q2 = _conv2_call(y1, sc1, sh1, w2c)
    sc2, sh2 = _stats_to_affine(s2, q2, g2, b2, M)

    o = _out_call(y2, sc2, sh2, xp)
    o = o.reshape(N, _H, _WP, C)[:, :, :_W, :]
    return jnp.transpose(o, (0, 3, 1, 2))


def kernel(x_nchw, w1m, g1, b1, w2m, g2, b2):
    return _forward(x_nchw, w1m, g1, b1, w2m, g2, b2)


# speedup vs baseline: 1.0933x; 1.0110x over previous
import jax
import jax.numpy as jnp
from jax.experimental import pallas as pl
from jax.experimental.pallas import tpu as pltpu

_BN_EPS = 1e-5
_H = 28
_W = 28
_WP = 32
_ROWS = 31 * _WP
_OUT_ROWS = _H * _WP


def _cparams():
    return pltpu.CompilerParams(
        dimension_semantics=("parallel",),
        vmem_limit_bytes=64 * 1024 * 1024,
    )


def _row_mask(shape):
    col = jax.lax.broadcasted_iota(jnp.int32, shape, 0) % _WP
    return col < _W


def _shifted_copies(xb):
    z1 = jnp.zeros((1, 128), dtype=xb.dtype)
    z2 = jnp.zeros((2, 128), dtype=xb.dtype)
    c1 = jnp.concatenate([xb[1:], z1], axis=0)
    c2 = jnp.concatenate([xb[2:], z2], axis=0)
    return [xb, c1, c2]


def _conv9(cs, w_ref):
    acc = jnp.zeros((_OUT_ROWS, 128), dtype=jnp.float32)
    for kx in range(3):
        ck = cs[kx]
        for ky in range(3):
            patch = ck[ky * _WP:ky * _WP + _OUT_ROWS]
            acc = acc + jnp.dot(patch, w_ref[kx, ky],
                                preferred_element_type=jnp.float32)
    return acc


def _conv1_kernel(xp_ref, w_ref, y_ref, ssum_ref, ssq_ref):
    cs = _shifted_copies(xp_ref[0])
    acc = _conv9(cs, w_ref)
    acc = jnp.where(_row_mask(acc.shape), acc, 0.0)
    y_ref[0] = acc.astype(jnp.bfloat16)
    ssum_ref[0] = jnp.sum(acc, axis=0, keepdims=True)
    ssq_ref[0] = jnp.sum(acc * acc, axis=0, keepdims=True)


def _affine_from_stats(s_ref, q_ref, g_ref, b_ref, m):
    total = jnp.sum(s_ref[:, 0, :], axis=0, keepdims=True)
    totsq = jnp.sum(q_ref[:, 0, :], axis=0, keepdims=True)
    mean = total / m
    var = totsq / m - mean * mean
    scale = g_ref[...] * jax.lax.rsqrt(var + _BN_EPS)
    shift = b_ref[...] - mean * scale
    return scale, shift


def _conv2_kernel(y1_ref, s_ref, q_ref, g_ref, b_ref, w_ref,
                  y2_ref, ssum_ref, ssq_ref):
    m = float(s_ref.shape[0] * _H * _W)
    sc, sh = _affine_from_stats(s_ref, q_ref, g_ref, b_ref, m)
    a = y1_ref[0].astype(jnp.float32) * sc + sh
    a = jnp.maximum(a, 0.0)
    a = jnp.where(_row_mask(a.shape), a, 0.0)
    ab = a.astype(jnp.bfloat16)

    def _placed(base):
        zt = jnp.zeros((base, 128), dtype=jnp.bfloat16)
        zb = jnp.zeros((_ROWS - _OUT_ROWS - base, 128), dtype=jnp.bfloat16)
        return jnp.concatenate([zt, ab, zb], axis=0)

    cs = [_placed(33), _placed(32), _placed(31)]
    acc = _conv9(cs, w_ref)
    acc = jnp.where(_row_mask(acc.shape), acc, 0.0)
    y2_ref[0] = acc.astype(jnp.bfloat16)
    ssum_ref[0] = jnp.sum(acc, axis=0, keepdims=True)
    ssq_ref[0] = jnp.sum(acc * acc, axis=0, keepdims=True)


def _out_kernel(y2_ref, s_ref, q_ref, g_ref, b_ref, xp_ref, o_ref):
    m = float(s_ref.shape[0] * _H * _W)
    sc, sh = _affine_from_stats(s_ref, q_ref, g_ref, b_ref, m)
    res = xp_ref[0][33:33 + _OUT_ROWS].astype(jnp.float32)
    o = y2_ref[0].astype(jnp.float32) * sc + sh + res
    o_ref[0] = jnp.maximum(o, 0.0)


def _conv1_call(xp, wc):
    N = xp.shape[0]
    flops = 2 * N * _H * _W * 9 * 128 * 128
    return pl.pallas_call(
        _conv1_kernel,
        out_shape=(jax.ShapeDtypeStruct((N, _OUT_ROWS, 128), jnp.bfloat16),
                   jax.ShapeDtypeStruct((N, 1, 128), jnp.float32),
                   jax.ShapeDtypeStruct((N, 1, 128), jnp.float32)),
        grid=(N,),
        in_specs=[pl.BlockSpec((1, _ROWS, 128), lambda n: (n, 0, 0)),
                  pl.BlockSpec((3, 3, 128, 128), lambda n: (0, 0, 0, 0))],
        out_specs=(pl.BlockSpec((1, _OUT_ROWS, 128), lambda n: (n, 0, 0)),
                   pl.BlockSpec((1, 1, 128), lambda n: (n, 0, 0)),
                   pl.BlockSpec((1, 1, 128), lambda n: (n, 0, 0))),
        compiler_params=_cparams(),
        cost_estimate=pl.CostEstimate(
            flops=flops, transcendentals=0,
            bytes_accessed=2 * (xp.size + wc.size) + 2 * N * _OUT_ROWS * 128),
    )(xp, wc)


def _conv2_call(y1, s1, q1, g1, b1, wc):
    N = y1.shape[0]
    flops = 2 * N * _H * _W * 9 * 128 * 128
    return pl.pallas_call(
        _conv2_kernel,
        out_shape=(jax.ShapeDtypeStruct((N, _OUT_ROWS, 128), jnp.bfloat16),
                   jax.ShapeDtypeStruct((N, 1, 128), jnp.float32),
                   jax.ShapeDtypeStruct((N, 1, 128), jnp.float32)),
        grid=(N,),
        in_specs=[pl.BlockSpec((1, _OUT_ROWS, 128), lambda n: (n, 0, 0)),
                  pl.BlockSpec((N, 1, 128), lambda n: (0, 0, 0)),
                  pl.BlockSpec((N, 1, 128), lambda n: (0, 0, 0)),
                  pl.BlockSpec((1, 128), lambda n: (0, 0)),
                  pl.BlockSpec((1, 128), lambda n: (0, 0)),
                  pl.BlockSpec((3, 3, 128, 128), lambda n: (0, 0, 0, 0))],
        out_specs=(pl.BlockSpec((1, _OUT_ROWS, 128), lambda n: (n, 0, 0)),
                   pl.BlockSpec((1, 1, 128), lambda n: (n, 0, 0)),
                   pl.BlockSpec((1, 1, 128), lambda n: (n, 0, 0))),
        compiler_params=_cparams(),
        cost_estimate=pl.CostEstimate(
            flops=flops, transcendentals=0,
            bytes_accessed=4 * N * _OUT_ROWS * 128),
    )(y1, s1, q1, g1, b1, wc)


def _out_call(y2, s2, q2, g2, b2, xp):
    N = y2.shape[0]
    return pl.pallas_call(
        _out_kernel,
        out_shape=jax.ShapeDtypeStruct((N, _OUT_ROWS, 128), jnp.float32),
        grid=(N,),
        in_specs=[pl.BlockSpec((1, _OUT_ROWS, 128), lambda n: (n, 0, 0)),
                  pl.BlockSpec((N, 1, 128), lambda n: (0, 0, 0)),
                  pl.BlockSpec((N, 1, 128), lambda n: (0, 0, 0)),
                  pl.BlockSpec((1, 128), lambda n: (0, 0)),
                  pl.BlockSpec((1, 128), lambda n: (0, 0)),
                  pl.BlockSpec((1, _ROWS, 128), lambda n: (n, 0, 0))],
        out_specs=pl.BlockSpec((1, _OUT_ROWS, 128), lambda n: (n, 0, 0)),
        compiler_params=_cparams(),
        cost_estimate=pl.CostEstimate(
            flops=4 * N * _OUT_ROWS * 128, transcendentals=0,
            bytes_accessed=10 * N * _OUT_ROWS * 128),
    )(y2, s2, q2, g2, b2, xp)


@jax.jit
def _forward(x_nchw, w1m, g1, b1, w2m, g2, b2):
    N, C, H, W = x_nchw.shape

    w1c = jnp.transpose(w1m.reshape(3, 3, C, C), (1, 0, 2, 3)).astype(jnp.bfloat16)
    w2c = jnp.transpose(w2m.reshape(3, 3, C, C), (1, 0, 2, 3)).astype(jnp.bfloat16)
    g1r, b1r = g1.reshape(1, C), b1.reshape(1, C)
    g2r, b2r = g2.reshape(1, C), b2.reshape(1, C)

    x_nhwc = jnp.transpose(x_nchw, (0, 2, 3, 1))
    xp = jnp.pad(x_nhwc, ((0, 0), (1, 2), (1, 3), (0, 0)))
    xp = xp.astype(jnp.bfloat16).reshape(N, _ROWS, C)

    y1, s1, q1 = _conv1_call(xp, w1c)
    y2, s2, q2 = _conv2_call(y1, s1, q1, g1r, b1r, w2c)
    o = _out_call(y2, s2, q2, g2r, b2r, xp)
    o = o.reshape(N, _H, _WP, C)[:, :, :_W, :]
    return jnp.transpose(o, (0, 3, 1, 2))


def kernel(x_nchw, w1m, g1, b1, w2m, g2, b2):
    return _forward(x_nchw, w1m, g1, b1, w2m, g2, b2)


# speedup vs baseline: 1.1620x; 1.0628x over previous
import jax
import jax.numpy as jnp
from jax.experimental import pallas as pl
from jax.experimental.pallas import tpu as pltpu

_BN_EPS = 1e-5
_H = 28
_W = 28
_WP = 32
_ROWS = 31 * _WP
_OUT_ROWS = _H * _WP


def _cparams():
    return pltpu.CompilerParams(
        dimension_semantics=("parallel",),
        vmem_limit_bytes=64 * 1024 * 1024,
    )


def _row_mask(shape):
    col = jax.lax.broadcasted_iota(jnp.int32, shape, 0) % _WP
    return col < _W


def _to_flat32(x_cm):
    xt = jnp.transpose(x_cm, (1, 0)).astype(jnp.bfloat16)
    x3 = jnp.pad(xt.reshape(_H, _W, 128), ((0, 0), (1, 3), (0, 0)))
    xf = x3.reshape(_OUT_ROWS, 128)
    zt = jnp.zeros((_WP, 128), dtype=jnp.bfloat16)
    zb = jnp.zeros((2 * _WP, 128), dtype=jnp.bfloat16)
    return jnp.concatenate([zt, xf, zb], axis=0)


def _shifted_copies(xb):
    z1 = jnp.zeros((1, 128), dtype=xb.dtype)
    z2 = jnp.zeros((2, 128), dtype=xb.dtype)
    c1 = jnp.concatenate([xb[1:], z1], axis=0)
    c2 = jnp.concatenate([xb[2:], z2], axis=0)
    return [xb, c1, c2]


def _conv9(cs, w_ref):
    acc = jnp.zeros((_OUT_ROWS, 128), dtype=jnp.float32)
    for kx in range(3):
        ck = cs[kx]
        for ky in range(3):
            patch = ck[ky * _WP:ky * _WP + _OUT_ROWS]
            wk = w_ref[ky * 3 + kx].astype(jnp.bfloat16)
            acc = acc + jnp.dot(patch, wk,
                                preferred_element_type=jnp.float32)
    return acc


def _affine_from_stats(s_ref, q_ref, g_ref, b_ref):
    m = float(s_ref.shape[0] * _H * _W)
    total = jnp.sum(s_ref[:, 0, :], axis=0, keepdims=True)
    totsq = jnp.sum(q_ref[:, 0, :], axis=0, keepdims=True)
    mean = total / m
    var = totsq / m - mean * mean
    scale = g_ref[...] * jax.lax.rsqrt(var + _BN_EPS)
    shift = b_ref[...] - mean * scale
    return scale, shift


def _conv1_kernel(xr_ref, w_ref, y_ref, ssum_ref, ssq_ref):
    cs = _shifted_copies(_to_flat32(xr_ref[0]))
    acc = _conv9(cs, w_ref)
    acc = jnp.where(_row_mask(acc.shape), acc, 0.0)
    y_ref[0] = acc.astype(jnp.bfloat16)
    ssum_ref[0] = jnp.sum(acc, axis=0, keepdims=True)
    ssq_ref[0] = jnp.sum(acc * acc, axis=0, keepdims=True)


def _conv2_kernel(y1_ref, s_ref, q_ref, g_ref, b_ref, w_ref,
                  y2_ref, ssum_ref, ssq_ref):
    sc, sh = _affine_from_stats(s_ref, q_ref, g_ref, b_ref)
    a = y1_ref[0].astype(jnp.float32) * sc + sh
    a = jnp.maximum(a, 0.0)
    a = jnp.where(_row_mask(a.shape), a, 0.0)
    ab = a.astype(jnp.bfloat16)

    def _placed(base):
        zt = jnp.zeros((base, 128), dtype=jnp.bfloat16)
        zb = jnp.zeros((_ROWS - _OUT_ROWS - base, 128), dtype=jnp.bfloat16)
        return jnp.concatenate([zt, ab, zb], axis=0)

    cs = [_placed(33), _placed(32), _placed(31)]
    acc = _conv9(cs, w_ref)
    acc = jnp.where(_row_mask(acc.shape), acc, 0.0)
    y2_ref[0] = acc.astype(jnp.bfloat16)
    ssum_ref[0] = jnp.sum(acc, axis=0, keepdims=True)
    ssq_ref[0] = jnp.sum(acc * acc, axis=0, keepdims=True)


def _out_kernel(y2_ref, s_ref, q_ref, g_ref, b_ref, xr_ref, o_ref):
    sc, sh = _affine_from_stats(s_ref, q_ref, g_ref, b_ref)
    v = y2_ref[0].astype(jnp.float32) * sc + sh
    v = v.reshape(_H, _WP, 128)[:, :_W, :].reshape(_H * _W, 128)
    vt = jnp.transpose(v, (1, 0))
    o_ref[0] = jnp.maximum(vt + xr_ref[0], 0.0)


def _conv1_call(xr, w1m):
    N = xr.shape[0]
    flops = 2 * N * _H * _W * 9 * 128 * 128
    return pl.pallas_call(
        _conv1_kernel,
        out_shape=(jax.ShapeDtypeStruct((N, _OUT_ROWS, 128), jnp.bfloat16),
                   jax.ShapeDtypeStruct((N, 1, 128), jnp.float32),
                   jax.ShapeDtypeStruct((N, 1, 128), jnp.float32)),
        grid=(N,),
        in_specs=[pl.BlockSpec((1, 128, _H * _W), lambda n: (n, 0, 0)),
                  pl.BlockSpec((9, 128, 128), lambda n: (0, 0, 0))],
        out_specs=(pl.BlockSpec((1, _OUT_ROWS, 128), lambda n: (n, 0, 0)),
                   pl.BlockSpec((1, 1, 128), lambda n: (n, 0, 0)),
                   pl.BlockSpec((1, 1, 128), lambda n: (n, 0, 0))),
        compiler_params=_cparams(),
        cost_estimate=pl.CostEstimate(
            flops=flops, transcendentals=0,
            bytes_accessed=4 * xr.size + 2 * N * _OUT_ROWS * 128),
    )(xr, w1m)


def _conv2_call(y1, s1, q1, g1, b1, w2m):
    N = y1.shape[0]
    flops = 2 * N * _H * _W * 9 * 128 * 128
    return pl.pallas_call(
        _conv2_kernel,
        out_shape=(jax.ShapeDtypeStruct((N, _OUT_ROWS, 128), jnp.bfloat16),
                   jax.ShapeDtypeStruct((N, 1, 128), jnp.float32),
                   jax.ShapeDtypeStruct((N, 1, 128), jnp.float32)),
        grid=(N,),
        in_specs=[pl.BlockSpec((1, _OUT_ROWS, 128), lambda n: (n, 0, 0)),
                  pl.BlockSpec((N, 1, 128), lambda n: (0, 0, 0)),
                  pl.BlockSpec((N, 1, 128), lambda n: (0, 0, 0)),
                  pl.BlockSpec((1, 128), lambda n: (0, 0)),
                  pl.BlockSpec((1, 128), lambda n: (0, 0)),
                  pl.BlockSpec((9, 128, 128), lambda n: (0, 0, 0))],
        out_specs=(pl.BlockSpec((1, _OUT_ROWS, 128), lambda n: (n, 0, 0)),
                   pl.BlockSpec((1, 1, 128), lambda n: (n, 0, 0)),
                   pl.BlockSpec((1, 1, 128), lambda n: (n, 0, 0))),
        compiler_params=_cparams(),
        cost_estimate=pl.CostEstimate(
            flops=flops, transcendentals=0,
            bytes_accessed=4 * N * _OUT_ROWS * 128),
    )(y1, s1, q1, g1, b1, w2m)


def _out_call(y2, s2, q2, g2, b2, xr):
    N = y2.shape[0]
    return pl.pallas_call(
        _out_kernel,
        out_shape=jax.ShapeDtypeStruct((N, 128, _H * _W), jnp.float32),
        grid=(N,),
        in_specs=[pl.BlockSpec((1, _OUT_ROWS, 128), lambda n: (n, 0, 0)),
                  pl.BlockSpec((N, 1, 128), lambda n: (0, 0, 0)),
                  pl.BlockSpec((N, 1, 128), lambda n: (0, 0, 0)),
                  pl.BlockSpec((1, 128), lambda n: (0, 0)),
                  pl.BlockSpec((1, 128), lambda n: (0, 0)),
                  pl.BlockSpec((1, 128, _H * _W), lambda n: (n, 0, 0))],
        out_specs=pl.BlockSpec((1, 128, _H * _W), lambda n: (n, 0, 0)),
        compiler_params=_cparams(),
        cost_estimate=pl.CostEstimate(
            flops=4 * N * _OUT_ROWS * 128, transcendentals=0,
            bytes_accessed=12 * N * _H * _W * 128),
    )(y2, s2, q2, g2, b2, xr)


@jax.jit
def _forward(x_nchw, w1m, g1, b1, w2m, g2, b2):
    N, C, H, W = x_nchw.shape
    xr = x_nchw.reshape(N, C, H * W)
    g1r, b1r = g1.reshape(1, C), b1.reshape(1, C)
    g2r, b2r = g2.reshape(1, C), b2.reshape(1, C)

    y1, s1, q1 = _conv1_call(xr, w1m)
    y2, s2, q2 = _conv2_call(y1, s1, q1, g1r, b1r, w2m)
    o = _out_call(y2, s2, q2, g2r, b2r, xr)
    return o.reshape(N, C, H, W)


def kernel(x_nchw, w1m, g1, b1, w2m, g2, b2):
    return _forward(x_nchw, w1m, g1, b1, w2m, g2, b2)


# speedup vs baseline: 1.3646x; 1.1743x over previous
import math

import jax
import jax.numpy as jnp
from jax.experimental import pallas as pl
from jax.experimental.pallas import tpu as pltpu

_BN_EPS = 1e-5
_H = 28
_W = 28
_WP = 32
_IMG = 31 * _WP
_OUT = _H * _WP
_B = 8


def _cparams():
    return pltpu.CompilerParams(
        dimension_semantics=("parallel",),
        vmem_limit_bytes=100 * 1024 * 1024,
    )


def _valid_mask(shape):
    r = jax.lax.broadcasted_iota(jnp.int32, shape, 0)
    return ((r % _WP) < _W) & ((r % _IMG) < _OUT)


def _shifted_copies(xb):
    rows = xb.shape[0]
    zt = jnp.zeros((66, 128), dtype=xb.dtype)
    ext = jnp.concatenate([xb, zt], axis=0)
    return [ext[0:rows + 64], ext[1:rows + 65], ext[2:rows + 66]]


def _conv9(cs, w_ref):
    rows = cs[0].shape[0] - 64
    acc = jnp.zeros((rows, 128), dtype=jnp.float32)
    for kx in range(3):
        ck = cs[kx]
        for ky in range(3):
            patch = ck[ky * _WP:ky * _WP + rows]
            wk = w_ref[ky * 3 + kx].astype(jnp.bfloat16)
            acc = acc + jnp.dot(patch, wk,
                                preferred_element_type=jnp.float32)
    return acc


def _affine_from_stats(s_ref, q_ref, g_ref, b_ref):
    m = float(s_ref.shape[0] * _H * _W)
    total = jnp.sum(s_ref[:, 0, :], axis=0, keepdims=True)
    totsq = jnp.sum(q_ref[:, 0, :], axis=0, keepdims=True)
    mean = total / m
    var = totsq / m - mean * mean
    scale = g_ref[...] * jax.lax.rsqrt(var + _BN_EPS)
    shift = b_ref[...] - mean * scale
    return scale, shift


def _finish(acc, y_ref, ssum_ref, ssq_ref):
    acc = jnp.where(_valid_mask(acc.shape), acc, 0.0)
    a3 = acc.reshape(y_ref.shape[0], _IMG, 128)
    y_ref[...] = a3[:, :_OUT, :].astype(jnp.bfloat16)
    ssum_ref[:, 0, :] = jnp.sum(a3, axis=1)
    ssq_ref[:, 0, :] = jnp.sum(a3 * a3, axis=1)


def _conv1_kernel(xr_ref, w_ref, y_ref, ssum_ref, ssq_ref):
    B = xr_ref.shape[0]
    xt = jnp.transpose(xr_ref[...], (0, 2, 1)).astype(jnp.bfloat16)
    x4 = jnp.pad(xt.reshape(B, _H, _W, 128),
                 ((0, 0), (0, 0), (1, 3), (0, 0)))
    xf = x4.reshape(B, _OUT, 128)
    zt = jnp.zeros((B, _WP, 128), dtype=jnp.bfloat16)
    zb = jnp.zeros((B, 2 * _WP, 128), dtype=jnp.bfloat16)
    xb = jnp.concatenate([zt, xf, zb], axis=1).reshape(B * _IMG, 128)
    acc = _conv9(_shifted_copies(xb), w_ref)
    _finish(acc, y_ref, ssum_ref, ssq_ref)


def _conv2_kernel(y1_ref, s_ref, q_ref, g_ref, b_ref, w_ref,
                  y2_ref, ssum_ref, ssq_ref):
    sc, sh = _affine_from_stats(s_ref, q_ref, g_ref, b_ref)
    a = y1_ref[...].astype(jnp.float32) * sc + sh
    a = jnp.maximum(a, 0.0)
    ab = a.astype(jnp.bfloat16)

    ab = jnp.where(_valid_mask_b(ab.shape), ab, jnp.bfloat16(0))

    B = y1_ref.shape[0]
    zt64 = jnp.zeros((64, 128), dtype=jnp.bfloat16)

    def _placed(base):
        zt = jnp.zeros((B, base, 128), dtype=jnp.bfloat16)
        zb = jnp.zeros((B, _IMG - _OUT - base, 128), dtype=jnp.bfloat16)
        flat = jnp.concatenate([zt, ab, zb], axis=1).reshape(B * _IMG, 128)
        return jnp.concatenate([flat, zt64], axis=0)

    cs = [_placed(33), _placed(32), _placed(31)]
    acc = _conv9(cs, w_ref)
    _finish(acc, y2_ref, ssum_ref, ssq_ref)


def _valid_mask_b(shape):
    r = jax.lax.broadcasted_iota(jnp.int32, shape, 1)
    return (r % _WP) < _W


def _out_kernel(y2_ref, s_ref, q_ref, g_ref, b_ref, xr_ref, o_ref):
    sc, sh = _affine_from_stats(s_ref, q_ref, g_ref, b_ref)
    B = y2_ref.shape[0]
    v = y2_ref[...].astype(jnp.float32) * sc + sh
    v = v.reshape(B, _H, _WP, 128)[:, :, :_W, :].reshape(B, _H * _W, 128)
    vt = jnp.transpose(v, (0, 2, 1))
    o_ref[...] = jnp.maximum(vt + xr_ref[...], 0.0)


def _conv1_call(xr, w1m):
    N = xr.shape[0]
    bb = math.gcd(N, _B)
    G = N // bb
    flops = 2 * N * _H * _W * 9 * 128 * 128
    return pl.pallas_call(
        _conv1_kernel,
        out_shape=(jax.ShapeDtypeStruct((N, _OUT, 128), jnp.bfloat16),
                   jax.ShapeDtypeStruct((N, 1, 128), jnp.float32),
                   jax.ShapeDtypeStruct((N, 1, 128), jnp.float32)),
        grid=(G,),
        in_specs=[pl.BlockSpec((bb, 128, _H * _W), lambda n: (n, 0, 0)),
                  pl.BlockSpec((9, 128, 128), lambda n: (0, 0, 0))],
        out_specs=(pl.BlockSpec((bb, _OUT, 128), lambda n: (n, 0, 0)),
                   pl.BlockSpec((bb, 1, 128), lambda n: (n, 0, 0)),
                   pl.BlockSpec((bb, 1, 128), lambda n: (n, 0, 0))),
        compiler_params=_cparams(),
        cost_estimate=pl.CostEstimate(
            flops=flops, transcendentals=0,
            bytes_accessed=4 * xr.size + 2 * N * _OUT * 128),
    )(xr, w1m)


def _conv2_call(y1, s1, q1, g1, b1, w2m):
    N = y1.shape[0]
    bb = math.gcd(N, _B)
    G = N // bb
    flops = 2 * N * _H * _W * 9 * 128 * 128
    return pl.pallas_call(
        _conv2_kernel,
        out_shape=(jax.ShapeDtypeStruct((N, _OUT, 128), jnp.bfloat16),
                   jax.ShapeDtypeStruct((N, 1, 128), jnp.float32),
                   jax.ShapeDtypeStruct((N, 1, 128), jnp.float32)),
        grid=(G,),
        in_specs=[pl.BlockSpec((bb, _OUT, 128), lambda n: (n, 0, 0)),
                  pl.BlockSpec((N, 1, 128), lambda n: (0, 0, 0)),
                  pl.BlockSpec((N, 1, 128), lambda n: (0, 0, 0)),
                  pl.BlockSpec((1, 128), lambda n: (0, 0)),
                  pl.BlockSpec((1, 128), lambda n: (0, 0)),
                  pl.BlockSpec((9, 128, 128), lambda n: (0, 0, 0))],
        out_specs=(pl.BlockSpec((bb, _OUT, 128), lambda n: (n, 0, 0)),
                   pl.BlockSpec((bb, 1, 128), lambda n: (n, 0, 0)),
                   pl.BlockSpec((bb, 1, 128), lambda n: (n, 0, 0))),
        compiler_params=_cparams(),
        cost_estimate=pl.CostEstimate(
            flops=flops, transcendentals=0,
            bytes_accessed=4 * N * _OUT * 128),
    )(y1, s1, q1, g1, b1, w2m)


def _out_call(y2, s2, q2, g2, b2, xr):
    N = y2.shape[0]
    bb = math.gcd(N, _B)
    G = N // bb
    return pl.pallas_call(
        _out_kernel,
        out_shape=jax.ShapeDtypeStruct((N, 128, _H * _W), jnp.float32),
        grid=(G,),
        in_specs=[pl.BlockSpec((bb, _OUT, 128), lambda n: (n, 0, 0)),
                  pl.BlockSpec((N, 1, 128), lambda n: (0, 0, 0)),
                  pl.BlockSpec((N, 1, 128), lambda n: (0, 0, 0)),
                  pl.BlockSpec((1, 128), lambda n: (0, 0)),
                  pl.BlockSpec((1, 128), lambda n: (0, 0)),
                  pl.BlockSpec((bb, 128, _H * _W), lambda n: (n, 0, 0))],
        out_specs=pl.BlockSpec((bb, 128, _H * _W), lambda n: (n, 0, 0)),
        compiler_params=_cparams(),
        cost_estimate=pl.CostEstimate(
            flops=4 * N * _OUT * 128, transcendentals=0,
            bytes_accessed=12 * N * _H * _W * 128),
    )(y2, s2, q2, g2, b2, xr)


@jax.jit
def _forward(x_nchw, w1m, g1, b1, w2m, g2, b2):
    N, C, H, W = x_nchw.shape
    xr = x_nchw.reshape(N, C, H * W)
    g1r, b1r = g1.reshape(1, C), b1.reshape(1, C)
    g2r, b2r = g2.reshape(1, C), b2.reshape(1, C)

    y1, s1, q1 = _conv1_call(xr, w1m)
    y2, s2, q2 = _conv2_call(y1, s1, q1, g1r, b1r, w2m)
    o = _out_call(y2, s2, q2, g2r, b2r, xr)
    return o.reshape(N, C, H, W)


def kernel(x_nchw, w1m, g1, b1, w2m, g2, b2):
    return _forward(x_nchw, w1m, g1, b1, w2m, g2, b2)


# speedup vs baseline: 1.6171x; 1.1851x over previous
import math

import jax
import jax.numpy as jnp
from jax.experimental import pallas as pl
from jax.experimental.pallas import tpu as pltpu

_BN_EPS = 1e-5
_H = 28
_W = 28
_WP = 32
_IMG = 31 * _WP
_OUT = _H * _WP
_B = 8


def _cparams():
    return pltpu.CompilerParams(
        dimension_semantics=("arbitrary",),
        vmem_limit_bytes=100 * 1024 * 1024,
    )


def _shifted_copies(xb):
    rows = xb.shape[0]
    zt = jnp.zeros((66, 128), dtype=xb.dtype)
    ext = jnp.concatenate([xb, zt], axis=0)
    return [ext[0:rows + 64], ext[1:rows + 65], ext[2:rows + 66]]


def _conv9(cs, w_ref):
    rows = cs[0].shape[0] - 64
    patches = [cs[k % 3][(k // 3) * _WP:(k // 3) * _WP + rows]
               for k in range(9)]
    p = jnp.concatenate(patches, axis=1)
    wk = w_ref[...].reshape(9 * 128, 128).astype(jnp.bfloat16)
    return jnp.dot(p, wk, preferred_element_type=jnp.float32)


def _affine_from_stats(s_ref, q_ref, g_ref, b_ref):
    m = float(s_ref.shape[0] * _H * _W)
    total = jnp.sum(s_ref[:, 0, :], axis=0, keepdims=True)
    totsq = jnp.sum(q_ref[:, 0, :], axis=0, keepdims=True)
    mean = total / m
    var = totsq / m - mean * mean
    scale = g_ref[...] * jax.lax.rsqrt(var + _BN_EPS)
    shift = b_ref[...] - mean * scale
    return scale, shift


def _finish(acc, y_ref, ssum_ref, ssq_ref):
    B = y_ref.shape[0]
    a3 = acc.reshape(B, _IMG, 128)
    y_ref[...] = a3[:, :_OUT, :].astype(jnp.bfloat16)
    a4 = acc.reshape(B, 31, _WP, 128)[:, :_H, :, :]
    s32 = jnp.sum(a4, axis=1)
    q32 = jnp.sum(a4 * a4, axis=1)
    ssum_ref[:, 0, :] = jnp.sum(s32[:, :_W, :], axis=1)
    ssq_ref[:, 0, :] = jnp.sum(q32[:, :_W, :], axis=1)


def _conv1_kernel(xr_ref, w_ref, y_ref, ssum_ref, ssq_ref):
    B = xr_ref.shape[0]
    xt = jnp.transpose(xr_ref[...].astype(jnp.bfloat16), (0, 2, 1))
    x4 = jnp.pad(xt.reshape(B, _H, _W, 128),
                 ((0, 0), (0, 0), (1, 3), (0, 0)))
    xf = x4.reshape(B, _OUT, 128)
    zt = jnp.zeros((B, _WP, 128), dtype=jnp.bfloat16)
    zb = jnp.zeros((B, 2 * _WP, 128), dtype=jnp.bfloat16)
    xb = jnp.concatenate([zt, xf, zb], axis=1).reshape(B * _IMG, 128)
    acc = _conv9(_shifted_copies(xb), w_ref)
    _finish(acc, y_ref, ssum_ref, ssq_ref)


def _conv2_kernel(y1_ref, s_ref, q_ref, g_ref, b_ref, w_ref,
                  y2_ref, ssum_ref, ssq_ref):
    sc, sh = _affine_from_stats(s_ref, q_ref, g_ref, b_ref)
    a = y1_ref[...].astype(jnp.float32) * sc + sh
    a = jnp.maximum(a, 0.0)

    B = y1_ref.shape[0]
    m32 = jax.lax.broadcasted_iota(jnp.int32, (_WP, 128), 0) < _W
    a4 = a.reshape(B, _H, _WP, 128)
    ab = jnp.where(m32[None, None, :, :], a4, 0.0).astype(jnp.bfloat16)
    ab = ab.reshape(B, _OUT, 128)
    zt64 = jnp.zeros((64, 128), dtype=jnp.bfloat16)

    def _placed(base):
        zt = jnp.zeros((B, base, 128), dtype=jnp.bfloat16)
        zb = jnp.zeros((B, _IMG - _OUT - base, 128), dtype=jnp.bfloat16)
        flat = jnp.concatenate([zt, ab, zb], axis=1).reshape(B * _IMG, 128)
        return jnp.concatenate([flat, zt64], axis=0)

    cs = [_placed(33), _placed(32), _placed(31)]
    acc = _conv9(cs, w_ref)
    _finish(acc, y2_ref, ssum_ref, ssq_ref)


def _out_kernel(y2_ref, s_ref, q_ref, g_ref, b_ref, xr_ref, o_ref):
    sc, sh = _affine_from_stats(s_ref, q_ref, g_ref, b_ref)
    B = y2_ref.shape[0]
    v = y2_ref[...].astype(jnp.float32) * sc + sh
    v = v.reshape(B, _H, _WP, 128)[:, :, :_W, :].reshape(B, _H * _W, 128)
    vt = jnp.transpose(v, (0, 2, 1))
    o_ref[...] = jnp.maximum(vt + xr_ref[...], 0.0)


def _conv1_call(xr, w1m):
    N = xr.shape[0]
    bb = math.gcd(N, _B)
    G = N // bb
    flops = 2 * N * _H * _W * 9 * 128 * 128
    return pl.pallas_call(
        _conv1_kernel,
        out_shape=(jax.ShapeDtypeStruct((N, _OUT, 128), jnp.bfloat16),
                   jax.ShapeDtypeStruct((N, 1, 128), jnp.float32),
                   jax.ShapeDtypeStruct((N, 1, 128), jnp.float32)),
        grid=(G,),
        in_specs=[pl.BlockSpec((bb, 128, _H * _W), lambda n: (n, 0, 0)),
                  pl.BlockSpec((9, 128, 128), lambda n: (0, 0, 0))],
        out_specs=(pl.BlockSpec((bb, _OUT, 128), lambda n: (n, 0, 0)),
                   pl.BlockSpec((bb, 1, 128), lambda n: (n, 0, 0)),
                   pl.BlockSpec((bb, 1, 128), lambda n: (n, 0, 0))),
        compiler_params=_cparams(),
        cost_estimate=pl.CostEstimate(
            flops=flops, transcendentals=0,
            bytes_accessed=4 * xr.size + 2 * N * _OUT * 128),
    )(xr, w1m)


def _conv2_call(y1, s1, q1, g1, b1, w2m):
    N = y1.shape[0]
    bb = math.gcd(N, _B)
    G = N // bb
    flops = 2 * N * _H * _W * 9 * 128 * 128
    return pl.pallas_call(
        _conv2_kernel,
        out_shape=(jax.ShapeDtypeStruct((N, _OUT, 128), jnp.bfloat16),
                   jax.ShapeDtypeStruct((N, 1, 128), jnp.float32),
                   jax.ShapeDtypeStruct((N, 1, 128), jnp.float32)),
        grid=(G,),
        in_specs=[pl.BlockSpec((bb, _OUT, 128), lambda n: (n, 0, 0)),
                  pl.BlockSpec((N, 1, 128), lambda n: (0, 0, 0)),
                  pl.BlockSpec((N, 1, 128), lambda n: (0, 0, 0)),
                  pl.BlockSpec((1, 128), lambda n: (0, 0)),
                  pl.BlockSpec((1, 128), lambda n: (0, 0)),
                  pl.BlockSpec((9, 128, 128), lambda n: (0, 0, 0))],
        out_specs=(pl.BlockSpec((bb, _OUT, 128), lambda n: (n, 0, 0)),
                   pl.BlockSpec((bb, 1, 128), lambda n: (n, 0, 0)),
                   pl.BlockSpec((bb, 1, 128), lambda n: (n, 0, 0))),
        compiler_params=_cparams(),
        cost_estimate=pl.CostEstimate(
            flops=flops, transcendentals=0,
            bytes_accessed=4 * N * _OUT * 128),
    )(y1, s1, q1, g1, b1, w2m)


def _out_call(y2, s2, q2, g2, b2, xr):
    N = y2.shape[0]
    bb = math.gcd(N, _B)
    G = N // bb
    return pl.pallas_call(
        _out_kernel,
        out_shape=jax.ShapeDtypeStruct((N, 128, _H * _W), jnp.float32),
        grid=(G,),
        in_specs=[pl.BlockSpec((bb, _OUT, 128), lambda n: (n, 0, 0)),
                  pl.BlockSpec((N, 1, 128), lambda n: (0, 0, 0)),
                  pl.BlockSpec((N, 1, 128), lambda n: (0, 0, 0)),
                  pl.BlockSpec((1, 128), lambda n: (0, 0)),
                  pl.BlockSpec((1, 128), lambda n: (0, 0)),
                  pl.BlockSpec((bb, 128, _H * _W), lambda n: (n, 0, 0))],
        out_specs=pl.BlockSpec((bb, 128, _H * _W), lambda n: (n, 0, 0)),
        compiler_params=_cparams(),
        cost_estimate=pl.CostEstimate(
            flops=4 * N * _OUT * 128, transcendentals=0,
            bytes_accessed=12 * N * _H * _W * 128),
    )(y2, s2, q2, g2, b2, xr)


@jax.jit
def _forward(x_nchw, w1m, g1, b1, w2m, g2, b2):
    N, C, H, W = x_nchw.shape
    xr = x_nchw.reshape(N, C, H * W)
    g1r, b1r = g1.reshape(1, C), b1.reshape(1, C)
    g2r, b2r = g2.reshape(1, C), b2.reshape(1, C)

    y1, s1, q1 = _conv1_call(xr, w1m)
    y2, s2, q2 = _conv2_call(y1, s1, q1, g1r, b1r, w2m)
    o = _out_call(y2, s2, q2, g2r, b2r, xr)
    return o.reshape(N, C, H, W)


def kernel(x_nchw, w1m, g1, b1, w2m, g2, b2):
    return _forward(x_nchw, w1m, g1, b1, w2m, g2, b2)


# speedup vs baseline: 1.6374x; 1.0125x over previous
import math

import jax
import jax.numpy as jnp
from jax.experimental import pallas as pl
from jax.experimental.pallas import tpu as pltpu

_BN_EPS = 1e-5
_H = 28
_W = 28
_WP = 32
_IMG = 31 * _WP
_OUT = _H * _WP
_B = 8


def _cparams():
    return pltpu.CompilerParams(
        dimension_semantics=("arbitrary",),
        vmem_limit_bytes=100 * 1024 * 1024,
    )


def _shifted_copies(xb):
    rows = xb.shape[0]
    zt = jnp.zeros((66, 128), dtype=xb.dtype)
    ext = jnp.concatenate([xb, zt], axis=0)
    return [ext[0:rows + 64], ext[1:rows + 65], ext[2:rows + 66]]


def _conv9(cs, w_ref, B):
    wk = w_ref[...].reshape(9 * 128, 128).astype(jnp.bfloat16)
    accs = []
    for i in range(B):
        base = i * _IMG
        patches = [cs[k % 3][base + (k // 3) * _WP:
                             base + (k // 3) * _WP + _OUT]
                   for k in range(9)]
        p = jnp.concatenate(patches, axis=1)
        accs.append(jnp.dot(p, wk, preferred_element_type=jnp.float32))
    return jnp.stack(accs, axis=0)


def _affine_from_stats(s_ref, q_ref, g_ref, b_ref):
    m = float(s_ref.shape[0] * _H * _W)
    total = jnp.sum(s_ref[:, 0, :], axis=0, keepdims=True)
    totsq = jnp.sum(q_ref[:, 0, :], axis=0, keepdims=True)
    mean = total / m
    var = totsq / m - mean * mean
    scale = g_ref[...] * jax.lax.rsqrt(var + _BN_EPS)
    shift = b_ref[...] - mean * scale
    return scale, shift


def _finish(acc, y_ref, ssum_ref, ssq_ref):
    B = y_ref.shape[0]
    y_ref[...] = acc.astype(jnp.bfloat16)
    a4 = acc.reshape(B, _H, _WP, 128)
    s32 = jnp.sum(a4, axis=1)
    q32 = jnp.sum(a4 * a4, axis=1)
    ssum_ref[:, 0, :] = jnp.sum(s32[:, :_W, :], axis=1)
    ssq_ref[:, 0, :] = jnp.sum(q32[:, :_W, :], axis=1)


def _conv1_kernel(xr_ref, w_ref, y_ref, ssum_ref, ssq_ref):
    B = xr_ref.shape[0]
    xt = jnp.transpose(xr_ref[...], (0, 2, 1))
    x4 = jnp.pad(xt.reshape(B, _H, _W, 128),
                 ((0, 0), (0, 0), (1, 3), (0, 0)))
    xf = x4.reshape(B, _OUT, 128)
    zt = jnp.zeros((B, _WP, 128), dtype=jnp.bfloat16)
    zb = jnp.zeros((B, 2 * _WP, 128), dtype=jnp.bfloat16)
    xb = jnp.concatenate([zt, xf, zb], axis=1).reshape(B * _IMG, 128)
    acc = _conv9(_shifted_copies(xb), w_ref, B)
    _finish(acc, y_ref, ssum_ref, ssq_ref)


def _conv2_kernel(y1_ref, s_ref, q_ref, g_ref, b_ref, w_ref,
                  y2_ref, ssum_ref, ssq_ref):
    sc, sh = _affine_from_stats(s_ref, q_ref, g_ref, b_ref)
    a = y1_ref[...].astype(jnp.float32) * sc + sh
    a = jnp.maximum(a, 0.0)

    B = y1_ref.shape[0]
    m32 = jax.lax.broadcasted_iota(jnp.int32, (_WP, 128), 0) < _W
    a4 = a.reshape(B, _H, _WP, 128)
    ab = jnp.where(m32[None, None, :, :], a4, 0.0).astype(jnp.bfloat16)
    ab = ab.reshape(B, _OUT, 128)
    zt64 = jnp.zeros((64, 128), dtype=jnp.bfloat16)

    def _placed(base):
        zt = jnp.zeros((B, base, 128), dtype=jnp.bfloat16)
        zb = jnp.zeros((B, _IMG - _OUT - base, 128), dtype=jnp.bfloat16)
        flat = jnp.concatenate([zt, ab, zb], axis=1).reshape(B * _IMG, 128)
        return jnp.concatenate([flat, zt64], axis=0)

    cs = [_placed(33), _placed(32), _placed(31)]
    acc = _conv9(cs, w_ref, B)
    _finish(acc, y2_ref, ssum_ref, ssq_ref)


def _out_kernel(y2_ref, s_ref, q_ref, g_ref, b_ref, xr_ref, o_ref):
    sc, sh = _affine_from_stats(s_ref, q_ref, g_ref, b_ref)
    B = y2_ref.shape[0]
    v = y2_ref[...].astype(jnp.float32) * sc + sh
    v = v.reshape(B, _H, _WP, 128)[:, :, :_W, :].reshape(B, _H * _W, 128)
    vt = jnp.transpose(v, (0, 2, 1))
    o_ref[...] = jnp.maximum(vt + xr_ref[...].astype(jnp.float32), 0.0)


def _conv1_call(xr, w1m):
    N = xr.shape[0]
    bb = math.gcd(N, _B)
    G = N // bb
    flops = 2 * N * _H * _W * 9 * 128 * 128
    return pl.pallas_call(
        _conv1_kernel,
        out_shape=(jax.ShapeDtypeStruct((N, _OUT, 128), jnp.bfloat16),
                   jax.ShapeDtypeStruct((N, 1, 128), jnp.float32),
                   jax.ShapeDtypeStruct((N, 1, 128), jnp.float32)),
        grid=(G,),
        in_specs=[pl.BlockSpec((bb, 128, _H * _W), lambda n: (n, 0, 0)),
                  pl.BlockSpec((9, 128, 128), lambda n: (0, 0, 0))],
        out_specs=(pl.BlockSpec((bb, _OUT, 128), lambda n: (n, 0, 0)),
                   pl.BlockSpec((bb, 1, 128), lambda n: (n, 0, 0)),
                   pl.BlockSpec((bb, 1, 128), lambda n: (n, 0, 0))),
        compiler_params=_cparams(),
        cost_estimate=pl.CostEstimate(
            flops=flops, transcendentals=0,
            bytes_accessed=4 * xr.size + 2 * N * _OUT * 128),
    )(xr, w1m)


def _conv2_call(y1, s1, q1, g1, b1, w2m):
    N = y1.shape[0]
    bb = math.gcd(N, _B)
    G = N // bb
    flops = 2 * N * _H * _W * 9 * 128 * 128
    return pl.pallas_call(
        _conv2_kernel,
        out_shape=(jax.ShapeDtypeStruct((N, _OUT, 128), jnp.bfloat16),
                   jax.ShapeDtypeStruct((N, 1, 128), jnp.float32),
                   jax.ShapeDtypeStruct((N, 1, 128), jnp.float32)),
        grid=(G,),
        in_specs=[pl.BlockSpec((bb, _OUT, 128), lambda n: (n, 0, 0)),
                  pl.BlockSpec((N, 1, 128), lambda n: (0, 0, 0)),
                  pl.BlockSpec((N, 1, 128), lambda n: (0, 0, 0)),
                  pl.BlockSpec((1, 128), lambda n: (0, 0)),
                  pl.BlockSpec((1, 128), lambda n: (0, 0)),
                  pl.BlockSpec((9, 128, 128), lambda n: (0, 0, 0))],
        out_specs=(pl.BlockSpec((bb, _OUT, 128), lambda n: (n, 0, 0)),
                   pl.BlockSpec((bb, 1, 128), lambda n: (n, 0, 0)),
                   pl.BlockSpec((bb, 1, 128), lambda n: (n, 0, 0))),
        compiler_params=_cparams(),
        cost_estimate=pl.CostEstimate(
            flops=flops, transcendentals=0,
            bytes_accessed=4 * N * _OUT * 128),
    )(y1, s1, q1, g1, b1, w2m)


def _out_call(y2, s2, q2, g2, b2, xr):
    N = y2.shape[0]
    bb = math.gcd(N, _B)
    G = N // bb
    return pl.pallas_call(
        _out_kernel,
        out_shape=jax.ShapeDtypeStruct((N, 128, _H * _W), jnp.float32),
        grid=(G,),
        in_specs=[pl.BlockSpec((bb, _OUT, 128), lambda n: (n, 0, 0)),
                  pl.BlockSpec((N, 1, 128), lambda n: (0, 0, 0)),
                  pl.BlockSpec((N, 1, 128), lambda n: (0, 0, 0)),
                  pl.BlockSpec((1, 128), lambda n: (0, 0)),
                  pl.BlockSpec((1, 128), lambda n: (0, 0)),
                  pl.BlockSpec((bb, 128, _H * _W), lambda n: (n, 0, 0))],
        out_specs=pl.BlockSpec((bb, 128, _H * _W), lambda n: (n, 0, 0)),
        compiler_params=_cparams(),
        cost_estimate=pl.CostEstimate(
            flops=4 * N * _OUT * 128, transcendentals=0,
            bytes_accessed=12 * N * _H * _W * 128),
    )(y2, s2, q2, g2, b2, xr)


@jax.jit
def _forward(x_nchw, w1m, g1, b1, w2m, g2, b2):
    N, C, H, W = x_nchw.shape
    xr = x_nchw.reshape(N, C, H * W).astype(jnp.bfloat16)
    g1r, b1r = g1.reshape(1, C), b1.reshape(1, C)
    g2r, b2r = g2.reshape(1, C), b2.reshape(1, C)

    y1, s1, q1 = _conv1_call(xr, w1m)
    y2, s2, q2 = _conv2_call(y1, s1, q1, g1r, b1r, w2m)
    o = _out_call(y2, s2, q2, g2r, b2r, xr)
    return o.reshape(N, C, H, W)


def kernel(x_nchw, w1m, g1, b1, w2m, g2, b2):
    return _forward(x_nchw, w1m, g1, b1, w2m, g2, b2)
